# baseline (speedup 1.0000x reference)
import functools

import jax
import jax.numpy as jnp
from jax.experimental import pallas as pl
from jax.experimental.pallas import tpu as pltpu

N = 8192
H = 1024
BLK = 256
SUB = 64
CH = 512
K_NEIGH = 6
NCH = N // CH


def _silu(x):
    return x * jax.nn.sigmoid(x)


def _rne(v):
    return v.astype(jnp.bfloat16).astype(jnp.float32)


def _fold(v, op):
    r = v[:, 0:128]
    for i in range(1, v.shape[1] // 128):
        r = op(r, v[:, i * 128:(i + 1) * 128])
    return r


def _body(
    coords_blk, coords_rows, w1, b1, w2, b2, wd1, bd1, wd2, bd2, out, d2s, mds
):
    xb = coords_blk[...]
    xr0 = coords_rows[0:1, :]
    xr1 = coords_rows[1:2, :]
    xr0r = _rne(xr0)
    xr1r = _rne(xr1)
    sqa = xr0 * xr0 + xr1 * xr1

    xbr = _rne(xb)
    w1r = _rne(w1[...])
    p = xbr[:, 0:1] * w1r[0:1, :] + xbr[:, 1:2] * w1r[1:2, :]
    h = _silu(p + b1[...])

    big = jnp.float32(jnp.inf)
    sqb_all = (xb[:, 0] * xb[:, 0] + xb[:, 1] * xb[:, 1])[:, None]
    sqb_hi = _rne(sqb_all)
    sqb_lo = sqb_all - sqb_hi
    ones_b = jnp.ones((BLK, 1), jnp.float32)
    amat = jnp.concatenate(
        [xb, sqb_hi, sqb_lo, ones_b, ones_b], axis=1
    )
    sqa_hi = _rne(sqa)
    sqa_lo = sqa - sqa_hi
    one_r = jnp.ones((1, N), jnp.float32)
    bmat = jnp.concatenate(
        [-2.0 * xr0, -2.0 * xr1, one_r, one_r, sqa_hi, sqa_lo], axis=0
    )
    for s in range(BLK // SUB):
        rs = slice(s * SUB, (s + 1) * SUB)
        xsr = xbr[rs]
        sqb = sqb_all[rs]
        d2s[...] = jax.lax.dot_general(
            amat[rs].astype(jnp.bfloat16),
            bmat.astype(jnp.bfloat16),
            (((1,), (0,)), ((), ())),
            preferred_element_type=jnp.float32,
        )
        accs = [jnp.full((SUB, 128), big, jnp.float32) for _ in range(4)]
        for c in range(N // 128):
            carry = d2s[:, c * 128:(c + 1) * 128]
            for i in range(3):
                hi = jnp.maximum(accs[i], carry)
                accs[i] = jnp.minimum(accs[i], carry)
                carry = hi
            accs[3] = jnp.minimum(accs[3], carry)
        a1, a2, a3, a4 = accs
        m = jnp.min(a1, axis=1)
        mins = [m]
        for _ in range(K_NEIGH):
            mb = m[:, None]
            cand = jnp.minimum(
                jnp.minimum(
                    jnp.where(a1 > mb, a1, big),
                    jnp.where(a2 > mb, a2, big),
                ),
                jnp.minimum(
                    jnp.where(a3 > mb, a3, big),
                    jnp.where(a4 > mb, a4, big),
                ),
            )
            m = jnp.min(cand, axis=1)
            mins.append(m)
        mb = m[:, None]
        acc_cnt = jnp.sum(
            (a1 <= mb).astype(jnp.float32)
            + (a2 <= mb).astype(jnp.float32)
            + (a3 <= mb).astype(jnp.float32),
            axis=1,
        )
        lane_full = jnp.sum((a4 <= mb).astype(jnp.float32), axis=1)
        cnt = jnp.where(lane_full > 0.0, -1.0, acc_cnt + lane_full)
        ssum = jnp.zeros((SUB,), jnp.float32)
        for m_p in mins[1:]:
            ssum = ssum + jnp.sqrt(jnp.maximum(m_p, 0.0))
        mds[rs, :] = (ssum * (1.0 / K_NEIGH))[:, None]

        def _slow_path(rs=rs):
            m = None
            for c in range(NCH):
                sl = slice(c * CH, (c + 1) * CH)
                pm = jnp.min(d2s[:, sl], axis=1)
                m = pm if m is None else jnp.minimum(m, pm)
            ssum = jnp.zeros((SUB,), jnp.float32)
            kcnt = jnp.zeros((SUB,), jnp.float32)
            for p_ in range(K_NEIGH + 1):
                last = p_ == K_NEIGH
                ccnt = jnp.zeros((SUB,), jnp.float32)
                nm = None
                for c in range(NCH):
                    sl = slice(c * CH, (c + 1) * CH)
                    v = d2s[:, sl]
                    eq = v == m[:, None]
                    ccnt = ccnt + jnp.sum(eq.astype(jnp.float32), axis=1)
                    if not last:
                        v2 = jnp.where(eq, big, v)
                        d2s[:, sl] = v2
                        pm = jnp.min(v2, axis=1)
                        nm = pm if nm is None else jnp.minimum(nm, pm)
                knew = kcnt + ccnt
                w = jnp.clip(
                    jnp.minimum(knew, 7.0) - jnp.maximum(kcnt, 1.0), 0.0, 6.0
                )
                ssum = ssum + jnp.where(
                    w > 0.0, w * jnp.sqrt(jnp.maximum(m, 0.0)), 0.0
                )
                kcnt = knew
                if not last:
                    m = nm
            mds[rs, :] = (ssum * (1.0 / K_NEIGH))[:, None]

        pl.when(jnp.any(cnt != 7.0))(_slow_path)
    mean_dist = mds[...]

    md = mean_dist
    pd = _rne(md) * _rne(wd1[...])[0:1, :]
    hd = _silu(pd + bd1[...])

    out1 = jax.lax.dot_general(
        h.astype(jnp.bfloat16),
        w2[...],
        (((1,), (0,)), ((), ())),
        preferred_element_type=jnp.float32,
    )
    out2 = jax.lax.dot_general(
        hd.astype(jnp.bfloat16),
        wd2[...],
        (((1,), (0,)), ((), ())),
        preferred_element_type=jnp.float32,
    )
    out[...] = (out1 + b2[...]) + (out2 + bd2[...])


@jax.jit
def _run(coordinates, coords_rows, W1, b1, W2, b2, Wd1, bd1, Wd2, bd2):
    grid = N // BLK
    return pl.pallas_call(
        _body,
        grid=(grid,),
        in_specs=[
            pl.BlockSpec((BLK, 2), lambda i: (i, 0)),
            pl.BlockSpec((8, N), lambda i: (0, 0)),
            pl.BlockSpec((2, H), lambda i: (0, 0)),
            pl.BlockSpec((H,), lambda i: (0,)),
            pl.BlockSpec((H, H), lambda i: (0, 0)),
            pl.BlockSpec((H,), lambda i: (0,)),
            pl.BlockSpec((1, H // 2), lambda i: (0, 0)),
            pl.BlockSpec((H // 2,), lambda i: (0,)),
            pl.BlockSpec((H // 2, H), lambda i: (0, 0)),
            pl.BlockSpec((H,), lambda i: (0,)),
        ],
        out_specs=pl.BlockSpec((BLK, H), lambda i: (i, 0)),
        out_shape=jax.ShapeDtypeStruct((N, H), jnp.float32),
        scratch_shapes=[
            pltpu.VMEM((SUB, N), jnp.float32),
            pltpu.VMEM((BLK, 1), jnp.float32),
        ],
    )(coordinates, coords_rows, W1, b1, W2, b2, Wd1, bd1, Wd2, bd2)


def kernel(coordinates, W1, b1, W2, b2, Wd1, bd1, Wd2, bd2, k):
    x = coordinates.astype(jnp.float32)
    xrows = jnp.zeros((8, N), jnp.float32).at[0:2, :].set(x.T)
    return _run(
        x, xrows, W1, b1, W2.astype(jnp.bfloat16), b2, Wd1, bd1,
        Wd2.astype(jnp.bfloat16), bd2,
    )


# speedup vs baseline: 2.0368x; 2.0368x over previous
import functools

import jax
import jax.numpy as jnp
from jax.experimental import pallas as pl
from jax.experimental.pallas import tpu as pltpu

N = 8192
H = 1024
BLK = 256
SUB = 128
CH = 512
K_NEIGH = 6
NCH = N // CH


def _silu(x):
    return x * jax.nn.sigmoid(x)


def _rne(v):
    return v.astype(jnp.bfloat16).astype(jnp.float32)


def _fold(v, op):
    r = v[:, 0:128]
    for i in range(1, v.shape[1] // 128):
        r = op(r, v[:, i * 128:(i + 1) * 128])
    return r


def _body(
    coords_blk, coords_rows, w1, b1, w2, b2, wd1, bd1, wd2, bd2, out, d2s, mds
):
    xb = coords_blk[...]
    xr0 = coords_rows[0:1, :]
    xr1 = coords_rows[1:2, :]
    xr0r = _rne(xr0)
    xr1r = _rne(xr1)
    sqa = xr0 * xr0 + xr1 * xr1

    xbr = _rne(xb)
    w1r = _rne(w1[...])
    p = xbr[:, 0:1] * w1r[0:1, :] + xbr[:, 1:2] * w1r[1:2, :]
    h = _silu(p + b1[...])

    big = jnp.float32(jnp.inf)
    sqb_all = xb[:, 0] * xb[:, 0] + xb[:, 1] * xb[:, 1]
    for s in range(BLK // SUB):
        rs = slice(s * SUB, (s + 1) * SUB)
        xsr = xbr[rs]
        sqb = sqb_all[rs][:, None]
        accs = [jnp.full((SUB, 128), big, jnp.float32) for _ in range(4)]
        for c in range(N // 128):
            sl = slice(c * 128, (c + 1) * 128)
            dot = xsr[:, 0:1] * xr0r[:, sl] + xsr[:, 1:2] * xr1r[:, sl]
            vs = (sqb + sqa[:, sl]) - 2.0 * dot
            carry = vs
            for i in range(3):
                hi = jnp.maximum(accs[i], carry)
                accs[i] = jnp.minimum(accs[i], carry)
                carry = hi
            accs[3] = jnp.minimum(accs[3], carry)
        a1, a2, a3, a4 = accs
        m = jnp.min(a1, axis=1)
        mins = [m]
        for _ in range(K_NEIGH):
            mb = m[:, None]
            cand = jnp.minimum(
                jnp.minimum(
                    jnp.where(a1 > mb, a1, big),
                    jnp.where(a2 > mb, a2, big),
                ),
                jnp.minimum(
                    jnp.where(a3 > mb, a3, big),
                    jnp.where(a4 > mb, a4, big),
                ),
            )
            m = jnp.min(cand, axis=1)
            mins.append(m)
        mb = m[:, None]
        acc_cnt = jnp.sum(
            (a1 <= mb).astype(jnp.float32)
            + (a2 <= mb).astype(jnp.float32)
            + (a3 <= mb).astype(jnp.float32),
            axis=1,
        )
        lane_full = jnp.sum((a4 <= mb).astype(jnp.float32), axis=1)
        cnt = jnp.where(lane_full > 0.0, -1.0, acc_cnt + lane_full)
        ssum = jnp.zeros((SUB,), jnp.float32)
        for m_p in mins[1:]:
            ssum = ssum + jnp.sqrt(jnp.maximum(m_p, 0.0))
        mds[rs, :] = (ssum * (1.0 / K_NEIGH))[:, None]

        def _slow_path(rs=rs, xsr=xsr, sqb=sqb):
            m = None
            for c in range(NCH):
                sl = slice(c * CH, (c + 1) * CH)
                dot = xsr[:, 0:1] * xr0r[:, sl] + xsr[:, 1:2] * xr1r[:, sl]
                v = (sqb + sqa[:, sl]) - 2.0 * dot
                d2s[:, sl] = v
                pm = jnp.min(v, axis=1)
                m = pm if m is None else jnp.minimum(m, pm)
            ssum = jnp.zeros((SUB,), jnp.float32)
            kcnt = jnp.zeros((SUB,), jnp.float32)
            for p_ in range(K_NEIGH + 1):
                last = p_ == K_NEIGH
                ccnt = jnp.zeros((SUB,), jnp.float32)
                nm = None
                for c in range(NCH):
                    sl = slice(c * CH, (c + 1) * CH)
                    v = d2s[:, sl]
                    eq = v == m[:, None]
                    ccnt = ccnt + jnp.sum(eq.astype(jnp.float32), axis=1)
                    if not last:
                        v2 = jnp.where(eq, big, v)
                        d2s[:, sl] = v2
                        pm = jnp.min(v2, axis=1)
                        nm = pm if nm is None else jnp.minimum(nm, pm)
                knew = kcnt + ccnt
                w = jnp.clip(
                    jnp.minimum(knew, 7.0) - jnp.maximum(kcnt, 1.0), 0.0, 6.0
                )
                ssum = ssum + jnp.where(
                    w > 0.0, w * jnp.sqrt(jnp.maximum(m, 0.0)), 0.0
                )
                kcnt = knew
                if not last:
                    m = nm
            mds[rs, :] = (ssum * (1.0 / K_NEIGH))[:, None]

        pl.when(jnp.any(cnt != 7.0))(_slow_path)
    mean_dist = mds[...]

    md = mean_dist
    pd = _rne(md) * _rne(wd1[...])[0:1, :]
    hd = _silu(pd + bd1[...])

    out1 = jax.lax.dot_general(
        h.astype(jnp.bfloat16),
        w2[...],
        (((1,), (0,)), ((), ())),
        preferred_element_type=jnp.float32,
    )
    out2 = jax.lax.dot_general(
        hd.astype(jnp.bfloat16),
        wd2[...],
        (((1,), (0,)), ((), ())),
        preferred_element_type=jnp.float32,
    )
    out[...] = (out1 + b2[...]) + (out2 + bd2[...])


@jax.jit
def _run(coordinates, coords_rows, W1, b1, W2, b2, Wd1, bd1, Wd2, bd2):
    grid = N // BLK
    return pl.pallas_call(
        _body,
        grid=(grid,),
        in_specs=[
            pl.BlockSpec((BLK, 2), lambda i: (i, 0)),
            pl.BlockSpec((8, N), lambda i: (0, 0)),
            pl.BlockSpec((2, H), lambda i: (0, 0)),
            pl.BlockSpec((H,), lambda i: (0,)),
            pl.BlockSpec((H, H), lambda i: (0, 0)),
            pl.BlockSpec((H,), lambda i: (0,)),
            pl.BlockSpec((1, H // 2), lambda i: (0, 0)),
            pl.BlockSpec((H // 2,), lambda i: (0,)),
            pl.BlockSpec((H // 2, H), lambda i: (0, 0)),
            pl.BlockSpec((H,), lambda i: (0,)),
        ],
        out_specs=pl.BlockSpec((BLK, H), lambda i: (i, 0)),
        out_shape=jax.ShapeDtypeStruct((N, H), jnp.float32),
        scratch_shapes=[
            pltpu.VMEM((SUB, N), jnp.float32),
            pltpu.VMEM((BLK, 1), jnp.float32),
        ],
    )(coordinates, coords_rows, W1, b1, W2, b2, Wd1, bd1, Wd2, bd2)


def kernel(coordinates, W1, b1, W2, b2, Wd1, bd1, Wd2, bd2, k):
    x = coordinates.astype(jnp.float32)
    xrows = jnp.zeros((8, N), jnp.float32).at[0:2, :].set(x.T)
    return _run(
        x, xrows, W1, b1, W2.astype(jnp.bfloat16), b2, Wd1, bd1,
        Wd2.astype(jnp.bfloat16), bd2,
    )
